# baseline (speedup 1.0000x reference)
import functools

import jax
import jax.numpy as jnp
from jax import lax
from jax.experimental import pallas as pl
from jax.experimental.pallas import tpu as pltpu
from jax.experimental.pallas import tpu_sc as plsc

N = 32768
D = 128
S = 2048
K = 10
NW = 16
SPW = S // NW
PPW = N // NW
NV = N // 16
INF = 0x7FFFFFFF

_mesh1 = functools.partial(
    plsc.VectorSubcoreMesh, core_axis_name="c", subcore_axis_name="s",
    num_cores=1)


def _samp_body(feats, idx1, randt, out,
               idx_full, focc, rc, randb, tb, orows, fb, fbuf):
    sid = lax.axis_index("s")
    s0 = sid * SPW
    iota = lax.iota(jnp.int32, 16)

    pltpu.sync_copy(idx1, idx_full)
    for g in range(SPW // 16):
        rc[pl.ds(g * 16, 16)] = jnp.zeros((16,), jnp.int32)
    pltpu.sync_copy(randt.at[pl.ds(sid * 8, 8)], randb)

    def it(v, carry):
        mk, mnb = carry
        x = idx_full[pl.ds(v * 16, 16)]
        iv = v * 16 + iota
        local = x - s0
        m = (local >= 0) & (local < SPW)
        occ, lastm = plsc.scan_count(x, mask=m)
        lsafe = local & (SPW - 1)
        old = plsc.load_gather(rc, [lsafe])
        r = old + occ - 1
        plsc.store_scatter(rc, [lsafe], old + occ, mask=m & lastm)
        m10 = m & (r < K)
        flat = lsafe * 16 + jnp.clip(r, 0, 15)
        plsc.store_scatter(focc, [flat], iv, mask=m10)
        key = x * N + iv
        mnb = jnp.minimum(mnb, jnp.where(x >= s0 + SPW, key, INF))
        mk = jnp.maximum(mk, key)
        return mk, mnb

    mk0 = jnp.full((16,), -1, jnp.int32)
    mnb0 = jnp.full((16,), INF, jnp.int32)
    mk, mnb = lax.fori_loop(0, NV, it, (mk0, mnb0))
    lastp = jnp.max(mk) & (N - 1)
    mnbs = jnp.min(mnb)

    carry0 = jnp.minimum(jnp.full((16,), INF, jnp.int32), mnbs)

    def fscan(gi, carry):
        g = SPW // 16 - 1 - gi
        jl = g * 16 + iota
        c = rc[pl.ds(g * 16, 16)]
        fo0 = plsc.load_gather(focc, [jl * 16])
        kj = jnp.where(c > 0, (s0 + jl) * N + fo0, INF)
        sm = -lax.rev(plsc.cummax(-lax.rev(kj, (0,))), (0,))
        smj = jnp.minimum(sm, carry)
        fv = jnp.where(smj < INF, smj & (N - 1), lastp)
        fb[pl.ds(g * 16, 16)] = fv
        return jnp.minimum(carry, jnp.min(kj))
    lax.fori_loop(0, SPW // 16, fscan, carry0)

    def tbuild(g, acc):
        jl = g * 16 + iota
        c = rc[pl.ds(g * 16, 16)]
        ml = jnp.maximum(c, 1)
        fbv = fb[pl.ds(g * 16, 16)]
        for k in range(K):
            rv = randb[g, pl.ds(k * 16, 16)]
            off = lax.rem(rv, ml)
            tv = plsc.load_gather(focc, [jl * 16 + off])
            tv = jnp.where(c == 0, fbv, tv)
            tb[k, pl.ds(g * 16, 16)] = tv
            orows[k, pl.ds(g * 16, 16)] = (s0 + jl) * K + k
        return acc
    lax.fori_loop(0, SPW // 16, tbuild, 0)

    for k in range(K):
        pltpu.sync_copy(feats.at[tb.at[k]], fbuf)
        pltpu.sync_copy(fbuf, out.at[orows.at[k]])


def _pool_body(feats, idx1, out, rc, fbuf, idxb, ct1, zbuf, recipb, iall,
               acc_sh, cnt_sh):
    sid = lax.axis_index("s")
    s0 = sid * SPW
    iota = lax.iota(jnp.int32, 16)

    def zb(r, acc):
        for c in range(8):
            zbuf[r, pl.ds(c * 16, 16)] = jnp.zeros((16,), jnp.float32)
        return acc
    lax.fori_loop(0, 8, zb, 0)

    def zct(r, acc):
        for c in range(8):
            ct1[r, pl.ds(c * 16, 16)] = jnp.zeros((16,), jnp.int32)
        return acc
    lax.fori_loop(0, NW, zct, 0)

    for t in range(SPW // 8):
        pltpu.sync_copy(zbuf, acc_sh.at[pl.ds(s0 + t * 8, 8)])

    @pl.when(sid == 0)
    def _():
        pltpu.sync_copy(ct1.at[pl.ds(0, 8)], cnt_sh.at[pl.ds(0, 8)])
        pltpu.sync_copy(ct1.at[pl.ds(0, 8)], cnt_sh.at[pl.ds(8, 8)])

    for t in range(NW):
        pltpu.sync_copy(idx1.at[pl.ds(sid * PPW + t * 128, 128)],
                        idxb.at[t])

    def cnt_row(r, acc):
        for c in range(8):
            x = idxb[r, pl.ds(c * 16, 16)]
            occ, lastm = plsc.scan_count(x)
            plsc.addupdate_scatter(
                ct1, [x >> 7, x & (SPW - 1)], occ, mask=lastm)
        return acc
    lax.fori_loop(0, NW, cnt_row, 0)

    iall[pl.ds(0, 16)] = iota
    plsc.subcore_barrier()

    pltpu.sync_copy(ct1, cnt_sh.at[iall], add=True)
    base = sid * PPW
    for t in range(PPW // 128):
        pltpu.sync_copy(feats.at[pl.ds(base + t * 128, 128)], fbuf)
        pltpu.sync_copy(fbuf, acc_sh.at[idxb.at[t]], add=True)

    plsc.subcore_barrier()

    pltpu.sync_copy(cnt_sh, idxb)
    pltpu.sync_copy(acc_sh.at[pl.ds(s0, SPW)], fbuf)
    for g in range(SPW // 16):
        c = idxb[sid, pl.ds(g * 16, 16)]
        cf = jnp.maximum(c, 1).astype(jnp.float32)
        recipb[pl.ds(g * 16, 16)] = 1.0 / cf

    def scale(j, acc):
        rv = plsc.load_gather(recipb, [jnp.full((16,), j, jnp.int32)])
        for c8 in range(8):
            fbuf[j, pl.ds(c8 * 16, 16)] = fbuf[j, pl.ds(c8 * 16, 16)] * rv
        return acc
    lax.fori_loop(0, SPW, scale, 0)
    pltpu.sync_copy(fbuf, out.at[pl.ds(s0, SPW)])


_samp = functools.partial(
    pl.kernel,
    out_type=jax.ShapeDtypeStruct((S * K, D), jnp.float32),
    mesh=_mesh1(),
    scratch_types=[
        pltpu.VMEM((N,), jnp.int32),
        pltpu.VMEM((SPW * 16,), jnp.int32),
        pltpu.VMEM((SPW,), jnp.int32),
        pltpu.VMEM((8, K * 16), jnp.int32),
        pltpu.VMEM((K, SPW), jnp.int32),
        pltpu.VMEM((K, SPW), jnp.int32),
        pltpu.VMEM((SPW,), jnp.int32),
        pltpu.VMEM((128, D), jnp.float32),
    ],
    compiler_params=pltpu.CompilerParams(needs_layout_passes=False),
)(_samp_body)

_pool = functools.partial(
    pl.kernel,
    out_type=jax.ShapeDtypeStruct((S, D), jnp.float32),
    mesh=_mesh1(),
    scratch_types=[
        pltpu.VMEM((SPW,), jnp.int32),
        pltpu.VMEM((128, D), jnp.float32),
        pltpu.VMEM((NW, 128), jnp.int32),
        pltpu.VMEM((NW, SPW), jnp.int32),
        pltpu.VMEM((8, D), jnp.float32),
        pltpu.VMEM((SPW,), jnp.float32),
        pltpu.VMEM((16,), jnp.int32),
        pltpu.VMEM_SHARED((S, D), jnp.float32),
        pltpu.VMEM_SHARED((NW, SPW), jnp.int32),
    ],
    compiler_params=pltpu.CompilerParams(needs_layout_passes=False),
)(_pool_body)


def kernel(raw_feats, raw2sp_idx):
    rand = jax.random.randint(jax.random.key(42), (S, K), 0, K)
    randt = rand.astype(jnp.int32).reshape(S // 16, 16, K)
    randt = randt.transpose(0, 2, 1).reshape(S // 16, K * 16)
    sampled = _samp(raw_feats, raw2sp_idx, randt)
    mean = _pool(raw_feats, raw2sp_idx)
    return jnp.concatenate(
        [sampled.reshape(S, K, D), mean[:, None, :]], axis=1)


# speedup vs baseline: 1.2322x; 1.2322x over previous
import functools

import jax
import jax.numpy as jnp
from jax import lax
from jax.experimental import pallas as pl
from jax.experimental.pallas import tpu as pltpu
from jax.experimental.pallas import tpu_sc as plsc

N = 32768
D = 128
S = 2048
K = 10
NW = 16
SPW = S // NW
PPW = N // NW
INF = 0x7FFFFFFF

_mesh = plsc.VectorSubcoreMesh(
    core_axis_name="c", subcore_axis_name="s", num_cores=2)


def _body(feats, idx1, randt, out,
          idx_full, focc, foccc, ct16, exb, rc, randb, tb, orows, fb,
          fbuf, fbuf2, idxb, ct1, zbuf, recipb, iall, sem0, sem1, sem2,
          acc_sh, cnt_sh):
    cid = lax.axis_index("c")
    sid = lax.axis_index("s")
    s0 = sid * SPW
    iota = lax.iota(jnp.int32, 16)

    @pl.when(cid == 0)
    def _():
        pltpu.sync_copy(idx1, idx_full)
        pltpu.sync_copy(randt.at[pl.ds(sid * 8, 8)], randb)

        def z16(j, acc):
            ct16[pl.ds(j * 16, 16)] = jnp.zeros((16,), jnp.int32)
            return acc
        lax.fori_loop(0, SPW, z16, 0)

        def passA(v, carry):
            mk, mnb = carry
            iv = iota * PPW + v
            x = plsc.load_gather(idx_full, [iv])
            local = x - s0
            m = (local >= 0) & (local < SPW)
            lsafe = local & (SPW - 1)
            flat = lsafe * 16 + iota
            old = plsc.load_gather(ct16, [flat])
            plsc.store_scatter(ct16, [flat], old + 1, mask=m)
            mc = m & (old < K)
            oldc = jnp.minimum(old, K - 1)
            plsc.store_scatter(foccc, [flat * K + oldc], iv, mask=mc)
            key = x * N + iv
            mnb = jnp.minimum(mnb, jnp.where(x >= s0 + SPW, key, INF))
            mk = jnp.maximum(mk, key)
            return mk, mnb

        mk0 = jnp.full((16,), -1, jnp.int32)
        mnb0 = jnp.full((16,), INF, jnp.int32)
        mk, mnb = lax.fori_loop(0, PPW, passA, (mk0, mnb0))
        lastp = jnp.max(mk) & (N - 1)
        mnbs = jnp.min(mnb)

        def pfx(j, acc):
            sl = pl.ds(j * 16, 16)
            c = ct16[sl]
            ex = plsc.cumsum(c) - c
            exb[sl] = ex
            return acc
        lax.fori_loop(0, SPW, pfx, 0)

        def brc(g, acc):
            fl = (g * 16 + iota) * 16 + 15
            rc[pl.ds(g * 16, 16)] = (
                plsc.load_gather(exb, [fl]) + plsc.load_gather(ct16, [fl]))
            return acc
        lax.fori_loop(0, SPW // 16, brc, 0)

        def merge(j, acc):
            base16 = exb[pl.ds(j * 16, 16)]
            cnt16 = ct16[pl.ds(j * 16, 16)]
            for rl in range(K):
                ival = plsc.load_gather(foccc, [(j * 16 + iota) * K + rl])
                g = base16 + rl
                valid = (rl < cnt16) & (g < K)
                plsc.store_scatter(
                    focc, [j * 16 + jnp.clip(g, 0, 15)], ival, mask=valid)
            return acc
        lax.fori_loop(0, SPW, merge, 0)

        carry0 = jnp.minimum(jnp.full((16,), INF, jnp.int32), mnbs)

        def fscan(gi, carry):
            g = SPW // 16 - 1 - gi
            jl = g * 16 + iota
            c = rc[pl.ds(g * 16, 16)]
            fo0 = plsc.load_gather(focc, [jl * 16])
            kj = jnp.where(c > 0, (s0 + jl) * N + fo0, INF)
            sm = -lax.rev(plsc.cummax(-lax.rev(kj, (0,))), (0,))
            smj = jnp.minimum(sm, carry)
            fv = jnp.where(smj < INF, smj & (N - 1), lastp)
            fb[pl.ds(g * 16, 16)] = fv
            return jnp.minimum(carry, jnp.min(kj))
        lax.fori_loop(0, SPW // 16, fscan, carry0)

        def tbuild(g, acc):
            jl = g * 16 + iota
            c = rc[pl.ds(g * 16, 16)]
            ml = jnp.maximum(c, 1)
            fbv = fb[pl.ds(g * 16, 16)]
            for k in range(K):
                rv = randb[g, pl.ds(k * 16, 16)]
                off = lax.rem(rv, ml)
                tv = plsc.load_gather(focc, [jl * 16 + off])
                tv = jnp.where(c == 0, fbv, tv)
                tb[k, pl.ds(g * 16, 16)] = tv
                orows[k, pl.ds(g * 16, 16)] = (s0 + jl) * (K + 1) + k
            return acc
        lax.fori_loop(0, SPW // 16, tbuild, 0)

        bufs = [fbuf, fbuf2]
        gsems = [sem0, sem1]
        d = pltpu.async_copy(feats.at[tb.at[0]], bufs[0], gsems[0])
        descs = [d]
        for k in range(K):
            if k + 1 < K:
                descs.append(pltpu.async_copy(
                    feats.at[tb.at[k + 1]], bufs[(k + 1) % 2],
                    gsems[(k + 1) % 2]))
            descs[k].wait()
            pltpu.sync_copy(bufs[k % 2], out.at[orows.at[k]])

    @pl.when(cid == 1)
    def _():
        def zb(r, acc):
            for c in range(8):
                zbuf[r, pl.ds(c * 16, 16)] = jnp.zeros((16,), jnp.float32)
            return acc
        lax.fori_loop(0, 8, zb, 0)

        def zct(r, acc):
            for c in range(8):
                ct1[r, pl.ds(c * 16, 16)] = jnp.zeros((16,), jnp.int32)
            return acc
        lax.fori_loop(0, NW, zct, 0)

        for t in range(SPW // 8):
            pltpu.sync_copy(zbuf, acc_sh.at[pl.ds(s0 + t * 8, 8)])

        @pl.when(sid == 0)
        def _():
            pltpu.sync_copy(ct1.at[pl.ds(0, 8)], cnt_sh.at[pl.ds(0, 8)])
            pltpu.sync_copy(ct1.at[pl.ds(0, 8)], cnt_sh.at[pl.ds(8, 8)])

        for t in range(NW):
            pltpu.sync_copy(idx1.at[pl.ds(sid * PPW + t * 128, 128)],
                            idxb.at[t])

        def cnt_row(r, acc):
            for c in range(8):
                x = idxb[r, pl.ds(c * 16, 16)]
                occ, lastm = plsc.scan_count(x)
                plsc.addupdate_scatter(
                    ct1, [x >> 7, x & (SPW - 1)], occ, mask=lastm)
            return acc
        lax.fori_loop(0, NW, cnt_row, 0)

        iall[pl.ds(0, 16)] = iota
        plsc.subcore_barrier()

        pltpu.sync_copy(ct1, cnt_sh.at[iall], add=True)
        base = sid * PPW

        bufs = [fbuf, fbuf2]
        gsems = [sem0, sem1]
        NT = PPW // 128
        gdescs = [pltpu.async_copy(
            feats.at[pl.ds(base, 128)], bufs[0], gsems[0])]
        sdescs = []
        for t in range(NT):
            if t + 1 < NT:
                if t >= 1:
                    sdescs[t - 1].wait()
                gdescs.append(pltpu.async_copy(
                    feats.at[pl.ds(base + (t + 1) * 128, 128)],
                    bufs[(t + 1) % 2], gsems[(t + 1) % 2]))
            gdescs[t].wait()
            sdescs.append(pltpu.async_copy(
                bufs[t % 2], acc_sh.at[idxb.at[t]], sem2, add=True))
        sdescs[NT - 2].wait()
        sdescs[NT - 1].wait()

        plsc.subcore_barrier()

        pltpu.sync_copy(cnt_sh, idxb)
        pltpu.sync_copy(acc_sh.at[pl.ds(s0, SPW)], fbuf)
        for g in range(SPW // 16):
            c = idxb[sid, pl.ds(g * 16, 16)]
            cf = jnp.maximum(c, 1).astype(jnp.float32)
            recipb[pl.ds(g * 16, 16)] = 1.0 / cf

        def scale(j, acc):
            rv = plsc.load_gather(recipb, [jnp.full((16,), j, jnp.int32)])
            for c8 in range(8):
                fbuf[j, pl.ds(c8 * 16, 16)] = (
                    fbuf[j, pl.ds(c8 * 16, 16)] * rv)
            return acc
        lax.fori_loop(0, SPW, scale, 0)
        for g in range(SPW // 16):
            orows[0, pl.ds(g * 16, 16)] = (s0 + g * 16 + iota) * (K + 1) + K
        pltpu.sync_copy(fbuf, out.at[orows.at[0]])


_hsm3 = functools.partial(
    pl.kernel,
    out_type=jax.ShapeDtypeStruct((S * (K + 1), D), jnp.float32),
    mesh=_mesh,
    scratch_types=[
        pltpu.VMEM((N,), jnp.int32),
        pltpu.VMEM((SPW * 16,), jnp.int32),
        pltpu.VMEM((SPW * 16 * K,), jnp.int32),
        pltpu.VMEM((SPW * 16,), jnp.int32),
        pltpu.VMEM((SPW * 16,), jnp.int32),
        pltpu.VMEM((SPW,), jnp.int32),
        pltpu.VMEM((8, K * 16), jnp.int32),
        pltpu.VMEM((K, SPW), jnp.int32),
        pltpu.VMEM((K, SPW), jnp.int32),
        pltpu.VMEM((SPW,), jnp.int32),
        pltpu.VMEM((128, D), jnp.float32),
        pltpu.VMEM((128, D), jnp.float32),
        pltpu.VMEM((NW, 128), jnp.int32),
        pltpu.VMEM((NW, SPW), jnp.int32),
        pltpu.VMEM((8, D), jnp.float32),
        pltpu.VMEM((SPW,), jnp.float32),
        pltpu.VMEM((16,), jnp.int32),
        pltpu.SemaphoreType.DMA,
        pltpu.SemaphoreType.DMA,
        pltpu.SemaphoreType.DMA,
        pltpu.VMEM_SHARED((S, D), jnp.float32),
        pltpu.VMEM_SHARED((NW, SPW), jnp.int32),
    ],
    compiler_params=pltpu.CompilerParams(needs_layout_passes=False),
)(_body)


def kernel(raw_feats, raw2sp_idx):
    rand = jax.random.randint(jax.random.key(42), (S, K), 0, K)
    randt = rand.astype(jnp.int32).reshape(S // 16, 16, K)
    randt = randt.transpose(0, 2, 1).reshape(S // 16, K * 16)
    out = _hsm3(raw_feats, raw2sp_idx, randt)
    return out.reshape(S, K + 1, D)


# speedup vs baseline: 1.4990x; 1.2166x over previous
import functools

import jax
import jax.numpy as jnp
from jax import lax
from jax.experimental import pallas as pl
from jax.experimental.pallas import tpu as pltpu
from jax.experimental.pallas import tpu_sc as plsc

N = 32768
D = 128
S = 2048
K = 10
NW = 16
SPW = S // NW
PPW = N // NW
INF = 0x7FFFFFFF

_mesh = plsc.VectorSubcoreMesh(
    core_axis_name="c", subcore_axis_name="s", num_cores=2)


def _body(feats, idx1, randt, out,
          idx_full, focc, rc, randb, tb, orows, fb,
          fbuf, fbuf2, idxb, ct1, zbuf, recipb, iall, sem0, sem1, sem2,
          acc_sh, cnt_sh):
    cid = lax.axis_index("c")
    sid = lax.axis_index("s")
    s0 = sid * SPW
    iota = lax.iota(jnp.int32, 16)

    @pl.when(cid == 0)
    def _():
        pltpu.sync_copy(idx1, idx_full)
        pltpu.sync_copy(randt.at[pl.ds(sid * 8, 8)], randb)

        def zrc(g, acc):
            rc[pl.ds(g * 16, 16)] = jnp.zeros((16,), jnp.int32)
            return acc
        lax.fori_loop(0, SPW // 16, zrc, 0)

        def it(v, carry):
            mk, mnb = carry
            x = idx_full[pl.ds(v * 16, 16)]
            iv = v * 16 + iota
            local = x - s0
            m = (local >= 0) & (local < SPW)
            occ, lastm = plsc.scan_count(x, mask=m)
            lsafe = local & (SPW - 1)
            old = plsc.load_gather(rc, [lsafe])
            r = old + occ - 1
            plsc.store_scatter(rc, [lsafe], old + occ, mask=m & lastm)
            m10 = m & (r < K)
            flat = lsafe * 16 + jnp.clip(r, 0, 15)
            plsc.store_scatter(focc, [flat], iv, mask=m10)
            key = x * N + iv
            mnb = jnp.minimum(mnb, jnp.where(x >= s0 + SPW, key, INF))
            mk = jnp.maximum(mk, key)
            return mk, mnb

        mk0 = jnp.full((16,), -1, jnp.int32)
        mnb0 = jnp.full((16,), INF, jnp.int32)
        mk, mnb = lax.fori_loop(0, N // 16, it, (mk0, mnb0))
        lastp = jnp.max(mk) & (N - 1)
        mnbs = jnp.min(mnb)

        carry0 = jnp.minimum(jnp.full((16,), INF, jnp.int32), mnbs)

        def fscan(gi, carry):
            g = SPW // 16 - 1 - gi
            jl = g * 16 + iota
            c = rc[pl.ds(g * 16, 16)]
            fo0 = plsc.load_gather(focc, [jl * 16])
            kj = jnp.where(c > 0, (s0 + jl) * N + fo0, INF)
            sm = -lax.rev(plsc.cummax(-lax.rev(kj, (0,))), (0,))
            smj = jnp.minimum(sm, carry)
            fv = jnp.where(smj < INF, smj & (N - 1), lastp)
            fb[pl.ds(g * 16, 16)] = fv
            return jnp.minimum(carry, jnp.min(kj))
        lax.fori_loop(0, SPW // 16, fscan, carry0)

        def tbuild(g, acc):
            jl = g * 16 + iota
            c = rc[pl.ds(g * 16, 16)]
            ml = jnp.maximum(c, 1)
            fbv = fb[pl.ds(g * 16, 16)]
            for k in range(K):
                rv = randb[g, pl.ds(k * 16, 16)]
                off = lax.rem(rv, ml)
                tv = plsc.load_gather(focc, [jl * 16 + off])
                tv = jnp.where(c == 0, fbv, tv)
                tb[k, pl.ds(g * 16, 16)] = tv
                orows[k, pl.ds(g * 16, 16)] = (s0 + jl) * (K + 1) + k
            return acc
        lax.fori_loop(0, SPW // 16, tbuild, 0)

        bufs = [fbuf, fbuf2]
        gsems = [sem0, sem1]
        d = pltpu.async_copy(feats.at[tb.at[0]], bufs[0], gsems[0])
        descs = [d]
        for k in range(K):
            if k + 1 < K:
                descs.append(pltpu.async_copy(
                    feats.at[tb.at[k + 1]], bufs[(k + 1) % 2],
                    gsems[(k + 1) % 2]))
            descs[k].wait()
            pltpu.sync_copy(bufs[k % 2], out.at[orows.at[k]])

    @pl.when(cid == 1)
    def _():
        def zb(r, acc):
            for c in range(8):
                zbuf[r, pl.ds(c * 16, 16)] = jnp.zeros((16,), jnp.float32)
            return acc
        lax.fori_loop(0, 8, zb, 0)

        def zct(r, acc):
            for c in range(8):
                ct1[r, pl.ds(c * 16, 16)] = jnp.zeros((16,), jnp.int32)
            return acc
        lax.fori_loop(0, NW, zct, 0)

        for t in range(SPW // 8):
            pltpu.sync_copy(zbuf, acc_sh.at[pl.ds(s0 + t * 8, 8)])

        @pl.when(sid == 0)
        def _():
            pltpu.sync_copy(ct1.at[pl.ds(0, 8)], cnt_sh.at[pl.ds(0, 8)])
            pltpu.sync_copy(ct1.at[pl.ds(0, 8)], cnt_sh.at[pl.ds(8, 8)])

        for t in range(NW):
            pltpu.sync_copy(idx1.at[pl.ds(sid * PPW + t * 128, 128)],
                            idxb.at[t])

        def cnt_row(r, acc):
            for c in range(8):
                x = idxb[r, pl.ds(c * 16, 16)]
                occ, lastm = plsc.scan_count(x)
                plsc.addupdate_scatter(
                    ct1, [x >> 7, x & (SPW - 1)], occ, mask=lastm)
            return acc
        lax.fori_loop(0, NW, cnt_row, 0)

        iall[pl.ds(0, 16)] = iota
        plsc.subcore_barrier()

        pltpu.sync_copy(ct1, cnt_sh.at[iall], add=True)
        base = sid * PPW

        bufs = [fbuf, fbuf2]
        gsems = [sem0, sem1]
        NT = PPW // 128
        gdescs = [pltpu.async_copy(
            feats.at[pl.ds(base, 128)], bufs[0], gsems[0])]
        sdescs = []
        for t in range(NT):
            if t + 1 < NT:
                if t >= 1:
                    sdescs[t - 1].wait()
                gdescs.append(pltpu.async_copy(
                    feats.at[pl.ds(base + (t + 1) * 128, 128)],
                    bufs[(t + 1) % 2], gsems[(t + 1) % 2]))
            gdescs[t].wait()
            sdescs.append(pltpu.async_copy(
                bufs[t % 2], acc_sh.at[idxb.at[t]], sem2, add=True))
        sdescs[NT - 2].wait()
        sdescs[NT - 1].wait()

        plsc.subcore_barrier()

        pltpu.sync_copy(cnt_sh, idxb)
        pltpu.sync_copy(acc_sh.at[pl.ds(s0, SPW)], fbuf)
        for g in range(SPW // 16):
            c = idxb[sid, pl.ds(g * 16, 16)]
            cf = jnp.maximum(c, 1).astype(jnp.float32)
            recipb[pl.ds(g * 16, 16)] = 1.0 / cf

        def scale(j, acc):
            rv = plsc.load_gather(recipb, [jnp.full((16,), j, jnp.int32)])
            for c8 in range(8):
                fbuf[j, pl.ds(c8 * 16, 16)] = (
                    fbuf[j, pl.ds(c8 * 16, 16)] * rv)
            return acc
        lax.fori_loop(0, SPW, scale, 0)
        for g in range(SPW // 16):
            orows[0, pl.ds(g * 16, 16)] = (s0 + g * 16 + iota) * (K + 1) + K
        pltpu.sync_copy(fbuf, out.at[orows.at[0]])


_hsm3 = functools.partial(
    pl.kernel,
    out_type=jax.ShapeDtypeStruct((S * (K + 1), D), jnp.float32),
    mesh=_mesh,
    scratch_types=[
        pltpu.VMEM((N,), jnp.int32),
        pltpu.VMEM((SPW * 16,), jnp.int32),
        pltpu.VMEM((SPW,), jnp.int32),
        pltpu.VMEM((8, K * 16), jnp.int32),
        pltpu.VMEM((K, SPW), jnp.int32),
        pltpu.VMEM((K, SPW), jnp.int32),
        pltpu.VMEM((SPW,), jnp.int32),
        pltpu.VMEM((128, D), jnp.float32),
        pltpu.VMEM((128, D), jnp.float32),
        pltpu.VMEM((NW, 128), jnp.int32),
        pltpu.VMEM((NW, SPW), jnp.int32),
        pltpu.VMEM((8, D), jnp.float32),
        pltpu.VMEM((SPW,), jnp.float32),
        pltpu.VMEM((16,), jnp.int32),
        pltpu.SemaphoreType.DMA,
        pltpu.SemaphoreType.DMA,
        pltpu.SemaphoreType.DMA,
        pltpu.VMEM_SHARED((S, D), jnp.float32),
        pltpu.VMEM_SHARED((NW, SPW), jnp.int32),
    ],
    compiler_params=pltpu.CompilerParams(needs_layout_passes=False),
)(_body)


def kernel(raw_feats, raw2sp_idx):
    rand = jax.random.randint(jax.random.key(42), (S, K), 0, K)
    randt = rand.astype(jnp.int32).reshape(S // 16, 16, K)
    randt = randt.transpose(0, 2, 1).reshape(S // 16, K * 16)
    out = _hsm3(raw_feats, raw2sp_idx, randt)
    return out.reshape(S, K + 1, D)
